# baseline (speedup 1.0000x reference)
import functools

import jax
import jax.numpy as jnp
from jax.experimental import pallas as pl
from jax.experimental.pallas import tpu as pltpu

B, A, IN, OUT, TOPK = 8, 2048, 128, 64, 16
SCALE = 8.0
BM = 256
NEG = -1e30


NSEG = 16
SEGW = A // NSEG
NCAND = NSEG * TOPK


def _gat_kernel(x_ref, conn_ref, wq_ref, wk_ref, wv_ref, gb_ref, out_ref,
                q_scr, k_scr, v_scr, s_scr, cand_scr, segmax_scr):
    i = pl.program_id(1)

    @pl.when(i == 0)
    def _():
        xb = x_ref[0]
        q_scr[...] = jax.lax.dot_general(
            xb, wq_ref[...], (((1,), (1,)), ((), ())),
            preferred_element_type=jnp.float32)
        k_scr[...] = jax.lax.dot_general(
            xb, wk_ref[...], (((1,), (1,)), ((), ())),
            preferred_element_type=jnp.float32)
        v_scr[...] = jax.lax.dot_general(
            xb, wv_ref[...], (((1,), (1,)), ((), ())),
            preferred_element_type=jnp.float32)

    qb = q_scr[pl.ds(i * BM, BM), :]
    s = jax.lax.dot_general(
        qb, k_scr[...], (((1,), (1,)), ((), ())),
        preferred_element_type=jnp.float32)
    s = s * (1.0 / SCALE) + conn_ref[0]

    s_scr[...] = s.reshape(BM, NSEG, SEGW)
    cand_scr[...] = jnp.full((TOPK, BM, NSEG), NEG, dtype=jnp.float32)

    segmax_scr[...] = jnp.max(s_scr[...], axis=-1)
    m1 = jnp.max(segmax_scr[...], axis=-1, keepdims=True)

    def cond(carry):
        k, done = carry
        return jnp.logical_and(k < TOPK, jnp.logical_not(done))

    def body(carry):
        k, _ = carry
        segmax = segmax_scr[...]
        cand_scr[k] = segmax
        w3 = s_scr[...]
        w3 = jnp.where(w3 == segmax[:, :, None], NEG, w3)
        s_scr[...] = w3
        segnew = jnp.max(w3, axis=-1)
        segmax_scr[...] = segnew
        rem = jnp.max(segnew, axis=-1,
                      keepdims=True)
        above = (cand_scr[...] >= rem[None, :, :]).astype(jnp.float32)
        cnt = jnp.sum(jnp.sum(above, axis=0), axis=-1, keepdims=True)
        done = jnp.all(cnt >= float(TOPK))
        return k + 1, done

    jax.lax.while_loop(cond, body, (jnp.int32(0), jnp.bool_(False)))

    def pbody(_, carry):
        del carry
        c = cand_scr[...]
        m = jnp.max(jnp.max(c, axis=0), axis=-1, keepdims=True)
        cand_scr[...] = jnp.where(c == m[None, :, :], NEG, c)
        return 0

    jax.lax.fori_loop(0, TOPK - 1, pbody, 0)
    t = jnp.max(jnp.max(cand_scr[...], axis=0), axis=-1, keepdims=True)

    w = jnp.where(s >= t, jnp.exp(s - m1), 0.0)
    z = jnp.sum(w, axis=-1, keepdims=True)
    o = jax.lax.dot_general(
        w, v_scr[...], (((1,), (0,)), ((), ())),
        preferred_element_type=jnp.float32)
    o = o / z

    mu = jnp.mean(o, axis=-1, keepdims=True)
    d = o - mu
    var = jnp.mean(d * d, axis=-1, keepdims=True)
    gamma = gb_ref[0:1, :]
    beta = gb_ref[1:2, :]
    out_ref[0] = d * jax.lax.rsqrt(var + 1e-5) * gamma + beta


@jax.jit
def kernel(x, connectivity, Wq, Wk, Wv, gamma, beta):
    gb = jnp.stack([gamma, beta], axis=0)
    grid = (B, A // BM)
    out = pl.pallas_call(
        _gat_kernel,
        grid=grid,
        in_specs=[
            pl.BlockSpec((1, A, IN), lambda b, i: (b, 0, 0)),
            pl.BlockSpec((1, BM, A), lambda b, i: (b, i, 0)),
            pl.BlockSpec((OUT, IN), lambda b, i: (0, 0)),
            pl.BlockSpec((OUT, IN), lambda b, i: (0, 0)),
            pl.BlockSpec((OUT, IN), lambda b, i: (0, 0)),
            pl.BlockSpec((2, OUT), lambda b, i: (0, 0)),
        ],
        out_specs=pl.BlockSpec((1, BM, OUT), lambda b, i: (b, i, 0)),
        out_shape=jax.ShapeDtypeStruct((B, A, OUT), jnp.float32),
        scratch_shapes=[
            pltpu.VMEM((A, OUT), jnp.float32),
            pltpu.VMEM((A, OUT), jnp.float32),
            pltpu.VMEM((A, OUT), jnp.float32),
            pltpu.VMEM((BM, NSEG, SEGW), jnp.float32),
            pltpu.VMEM((TOPK, BM, NSEG), jnp.float32),
            pltpu.VMEM((BM, NSEG), jnp.float32),
        ],
        compiler_params=pltpu.CompilerParams(
            dimension_semantics=("arbitrary", "arbitrary"),
        ),
    )(x, connectivity, Wq, Wk, Wv, gb)
    return out


# speedup vs baseline: 2.8166x; 2.8166x over previous
import functools

import jax
import jax.numpy as jnp
from jax.experimental import pallas as pl
from jax.experimental.pallas import tpu as pltpu

B, A, IN, OUT, TOPK = 8, 2048, 128, 64, 16
SCALE = 8.0
BM = 256
NEG = -1e30


def _gat_kernel(x_ref, conn_ref, wq_ref, wk_ref, wv_ref, gb_ref, out_ref,
                q_scr, k_scr, v_scr):
    i = pl.program_id(1)

    @pl.when(i == 0)
    def _():
        xb = x_ref[0]
        q_scr[...] = jax.lax.dot_general(
            xb, wq_ref[...], (((1,), (1,)), ((), ())),
            preferred_element_type=jnp.float32)
        k_scr[...] = jax.lax.dot_general(
            xb, wk_ref[...], (((1,), (1,)), ((), ())),
            preferred_element_type=jnp.float32)
        v_scr[...] = jax.lax.dot_general(
            xb, wv_ref[...], (((1,), (1,)), ((), ())),
            preferred_element_type=jnp.float32)

    qb = q_scr[pl.ds(i * BM, BM), :]
    s = jax.lax.dot_general(
        qb, k_scr[...], (((1,), (1,)), ((), ())),
        preferred_element_type=jnp.float32)
    s = s * (1.0 / SCALE) + conn_ref[0]

    m1 = jnp.max(s, axis=-1, keepdims=True)

    def body(_, m):
        return jnp.max(jnp.where(s < m, s, NEG), axis=-1, keepdims=True)

    t = jax.lax.fori_loop(0, TOPK - 1, body, m1)

    w = jnp.where(s >= t, jnp.exp(s - m1), 0.0)
    z = jnp.sum(w, axis=-1, keepdims=True)
    o = jax.lax.dot_general(
        w, v_scr[...], (((1,), (0,)), ((), ())),
        preferred_element_type=jnp.float32)
    o = o / z

    mu = jnp.mean(o, axis=-1, keepdims=True)
    d = o - mu
    var = jnp.mean(d * d, axis=-1, keepdims=True)
    gamma = gb_ref[0:1, :]
    beta = gb_ref[1:2, :]
    out_ref[0] = d * jax.lax.rsqrt(var + 1e-5) * gamma + beta


@jax.jit
def kernel(x, connectivity, Wq, Wk, Wv, gamma, beta):
    gb = jnp.stack([gamma, beta], axis=0)
    grid = (B, A // BM)
    out = pl.pallas_call(
        _gat_kernel,
        grid=grid,
        in_specs=[
            pl.BlockSpec((1, A, IN), lambda b, i: (b, 0, 0)),
            pl.BlockSpec((1, BM, A), lambda b, i: (b, i, 0)),
            pl.BlockSpec((OUT, IN), lambda b, i: (0, 0)),
            pl.BlockSpec((OUT, IN), lambda b, i: (0, 0)),
            pl.BlockSpec((OUT, IN), lambda b, i: (0, 0)),
            pl.BlockSpec((2, OUT), lambda b, i: (0, 0)),
        ],
        out_specs=pl.BlockSpec((1, BM, OUT), lambda b, i: (b, i, 0)),
        out_shape=jax.ShapeDtypeStruct((B, A, OUT), jnp.float32),
        scratch_shapes=[
            pltpu.VMEM((A, OUT), jnp.float32),
            pltpu.VMEM((A, OUT), jnp.float32),
            pltpu.VMEM((A, OUT), jnp.float32),
        ],
        compiler_params=pltpu.CompilerParams(
            dimension_semantics=("arbitrary", "arbitrary"),
        ),
    )(x, connectivity, Wq, Wk, Wv, gb)
    return out


# speedup vs baseline: 3.4409x; 1.2217x over previous
import functools

import jax
import jax.numpy as jnp
from jax.experimental import pallas as pl
from jax.experimental.pallas import tpu as pltpu

B, A, IN, OUT, TOPK = 8, 2048, 128, 64, 16
SCALE = 8.0
BM = 256
NEG = -1e30


NSL = 16
SLW = A // NSL


def _gat_kernel(x_ref, conn_ref, wq_ref, wk_ref, wv_ref, gb_ref, out_ref,
                q_scr, k_scr, v_scr, cand_scr, segm_scr, t_scr):
    i = pl.program_id(1)

    @pl.when(i == 0)
    def _():
        xb = x_ref[0]
        q_scr[...] = jax.lax.dot_general(
            xb, wq_ref[...], (((1,), (1,)), ((), ())),
            preferred_element_type=jnp.float32)
        k_scr[...] = jax.lax.dot_general(
            xb, wk_ref[...], (((1,), (1,)), ((), ())),
            preferred_element_type=jnp.float32)
        v_scr[...] = jax.lax.dot_general(
            xb, wv_ref[...], (((1,), (1,)), ((), ())),
            preferred_element_type=jnp.float32)

    qb = q_scr[pl.ds(i * BM, BM), :]
    s = jax.lax.dot_general(
        qb, k_scr[...], (((1,), (1,)), ((), ())),
        preferred_element_type=jnp.float32)
    s = s * (1.0 / SCALE) + conn_ref[0]

    slices = [s[:, k * SLW:(k + 1) * SLW] for k in range(NSL)]

    def class_max(vals):
        m = vals[0]
        for v in vals[1:]:
            m = jnp.maximum(m, v)
        return m

    def peel(segm, j):
        cand_scr[j] = segm
        return class_max([jnp.where(sl < segm, sl, NEG) for sl in slices])

    def check(segm, k):
        r = jnp.max(segm, axis=-1, keepdims=True)
        above = jnp.zeros((BM, SLW), dtype=jnp.float32)
        for j in range(k):
            above = above + (cand_scr[j] >= r).astype(jnp.float32)
        cnt = jnp.sum(above, axis=-1, keepdims=True)
        return jnp.all(cnt >= float(TOPK))

    def finish(k):
        def fbody(_, m):
            vals = [cand_scr[j] for j in range(k)]
            nm = class_max([jnp.where(c < m, c, NEG) for c in vals])
            return jnp.max(nm, axis=-1, keepdims=True)

        t_scr[...] = jax.lax.fori_loop(0, TOPK - 1, fbody, m1)

    segm = class_max(slices)
    m1 = jnp.max(segm, axis=-1, keepdims=True)

    for j in range(4):
        segm = peel(segm, j)
    done1 = check(segm, 4)
    segm_scr[...] = segm

    @pl.when(done1)
    def _():
        finish(4)

    @pl.when(jnp.logical_not(done1))
    def _():
        sg = segm_scr[...]
        for j in range(4, 6):
            sg = peel(sg, j)
        done2 = check(sg, 6)
        segm_scr[...] = sg

        @pl.when(done2)
        def _():
            finish(6)

        @pl.when(jnp.logical_not(done2))
        def _():
            sg2 = segm_scr[...]
            for j in range(6, TOPK):
                sg2 = peel(sg2, j)
            finish(TOPK)

    t = t_scr[...]

    w = jnp.where(s >= t, jnp.exp(s - m1), 0.0)
    z = jnp.sum(w, axis=-1, keepdims=True)
    o = jax.lax.dot_general(
        w, v_scr[...], (((1,), (0,)), ((), ())),
        preferred_element_type=jnp.float32)
    o = o / z

    mu = jnp.mean(o, axis=-1, keepdims=True)
    d = o - mu
    var = jnp.mean(d * d, axis=-1, keepdims=True)
    gamma = gb_ref[0:1, :]
    beta = gb_ref[1:2, :]
    out_ref[0] = d * jax.lax.rsqrt(var + 1e-5) * gamma + beta


@jax.jit
def kernel(x, connectivity, Wq, Wk, Wv, gamma, beta):
    gb = jnp.stack([gamma, beta], axis=0)
    grid = (B, A // BM)
    out = pl.pallas_call(
        _gat_kernel,
        grid=grid,
        in_specs=[
            pl.BlockSpec((1, A, IN), lambda b, i: (b, 0, 0)),
            pl.BlockSpec((1, BM, A), lambda b, i: (b, i, 0)),
            pl.BlockSpec((OUT, IN), lambda b, i: (0, 0)),
            pl.BlockSpec((OUT, IN), lambda b, i: (0, 0)),
            pl.BlockSpec((OUT, IN), lambda b, i: (0, 0)),
            pl.BlockSpec((2, OUT), lambda b, i: (0, 0)),
        ],
        out_specs=pl.BlockSpec((1, BM, OUT), lambda b, i: (b, i, 0)),
        out_shape=jax.ShapeDtypeStruct((B, A, OUT), jnp.float32),
        scratch_shapes=[
            pltpu.VMEM((A, OUT), jnp.float32),
            pltpu.VMEM((A, OUT), jnp.float32),
            pltpu.VMEM((A, OUT), jnp.float32),
            pltpu.VMEM((TOPK, BM, SLW), jnp.float32),
            pltpu.VMEM((BM, SLW), jnp.float32),
            pltpu.VMEM((BM, 1), jnp.float32),
        ],
        compiler_params=pltpu.CompilerParams(
            dimension_semantics=("arbitrary", "arbitrary"),
        ),
    )(x, connectivity, Wq, Wk, Wv, gb)
    return out
